# baseline (speedup 1.0000x reference)
import functools

import jax
import jax.numpy as jnp
from jax import lax
from jax.experimental import pallas as pl
from jax.experimental.pallas import tpu as pltpu
from jax.experimental.pallas import tpu_sc as plsc

NC = 2
NS = 16
CH = 128
D = 128

N_OUT = 10240
DEG_PAD = N_OUT
PAD_ROWS = 128


def _edge_plan(e_tot):
    cps = -(-e_tot // (NS * NC * CH))
    if cps % 2:
        cps += 1
    e_pad = cps * NS * NC * CH
    return cps, e_pad


def _make_deg_kernel(chalf):
    rows_pt = DEG_PAD // NS
    mesh = plsc.VectorSubcoreMesh(core_axis_name="c", subcore_axis_name="s",
                                  num_cores=NC, num_subcores=NS)

    @functools.partial(
        pl.kernel,
        out_type=jax.ShapeDtypeStruct((NC, DEG_PAD), jnp.float32),
        mesh=mesh,
        scratch_types=[
            pltpu.VMEM((chalf, CH), jnp.int32),
            pltpu.VMEM((CH,), jnp.float32),
            pltpu.VMEM((rows_pt,), jnp.float32),
            pltpu.VMEM_SHARED((DEG_PAD,), jnp.float32),
        ],
    )
    def deg_kernel(dst_hbm, out_hbm, idx_v, ones_v, zb_v, acc):
        c = lax.axis_index("c")
        s = lax.axis_index("s")
        wid = s * NC + c
        one = jnp.ones((16,), jnp.float32)
        zero = jnp.zeros((16,), jnp.float32)
        for i in range(CH // 16):
            ones_v[pl.ds(i * 16, 16)] = one

        def zfill(r, carry):
            zb_v[pl.ds(r * 16, 16)] = zero
            return carry

        lax.fori_loop(0, rows_pt // 16, zfill, 0)
        pltpu.sync_copy(zb_v, acc.at[pl.ds(s * rows_pt, rows_pt)])
        plsc.subcore_barrier()

        pltpu.sync_copy(dst_hbm.at[wid], idx_v)

        def body(j, carry):
            pltpu.sync_copy(ones_v, acc.at[idx_v.at[j]], add=True)
            return carry

        lax.fori_loop(0, chalf, body, 0)
        plsc.subcore_barrier()
        pltpu.sync_copy(acc.at[pl.ds(s * rows_pt, rows_pt)],
                        out_hbm.at[c, pl.ds(s * rows_pt, rows_pt)])

    return deg_kernel


def _make_mp_kernel(cps):
    rows_pt = N_OUT // NS
    epw = cps * CH
    mesh = plsc.VectorSubcoreMesh(core_axis_name="c", subcore_axis_name="s",
                                  num_cores=NC, num_subcores=NS)

    @functools.partial(
        pl.kernel,
        out_type=jax.ShapeDtypeStruct((NC, N_OUT, D), jnp.float32),
        mesh=mesh,
        scratch_types=[
            pltpu.VMEM((cps, CH), jnp.int32),
            pltpu.VMEM((2, CH), jnp.int32),
            pltpu.VMEM((CH, D), jnp.float32),
            pltpu.VMEM((CH, D), jnp.float32),
            pltpu.VMEM_SHARED((N_OUT, D), jnp.float32),
            pltpu.SemaphoreType.DMA,
            pltpu.SemaphoreType.DMA,
            pltpu.SemaphoreType.DMA,
            pltpu.SemaphoreType.DMA,
            pltpu.SemaphoreType.DMA,
            pltpu.SemaphoreType.DMA,
        ],
    )
    def mp_kernel(g_hbm, src_hbm, dst_hbm, out_hbm,
                  dst_v, sidx, rows_a, rows_b, acc, i0, i1, g0, g1, s0, s1):
        c = lax.axis_index("c")
        s = lax.axis_index("s")
        zero = jnp.zeros((16,), jnp.float32)
        base = (s * NC + c) * epw

        pltpu.sync_copy(dst_hbm.at[s, c], dst_v)

        def zfill(r, carry):
            for i in range(D // 16):
                rows_a[r, pl.ds(i * 16, 16)] = zero
            return carry

        lax.fori_loop(0, CH, zfill, 0)
        for t in range(rows_pt // CH):
            pltpu.sync_copy(rows_a, acc.at[pl.ds(s * rows_pt + t * CH, CH), :])
        plsc.subcore_barrier()

        def sload(chunk, slot, sem):
            pltpu.async_copy(src_hbm.at[pl.ds(base + chunk * CH, CH)],
                             sidx.at[slot], sem)

        def sload_wait(chunk, slot, sem):
            pltpu.make_async_copy(src_hbm.at[pl.ds(base + chunk * CH, CH)],
                                  sidx.at[slot], sem).wait()

        sload(0, 0, i0)
        sload(1, 1, i1)
        sload_wait(0, 0, i0)
        pltpu.async_copy(g_hbm.at[sidx.at[0]], rows_a, g0)
        sload_wait(1, 1, i1)
        pltpu.async_copy(g_hbm.at[sidx.at[1]], rows_b, g1)

        def pair(p, carry):
            c0 = 2 * p
            c1 = c0 + 1
            pltpu.make_async_copy(g_hbm.at[sidx.at[0]], rows_a, g0).wait()
            pltpu.async_copy(rows_a, acc.at[dst_v.at[c0]], s0, add=True)
            sload(c0 + 2, 0, i0)
            pltpu.make_async_copy(g_hbm.at[sidx.at[1]], rows_b, g1).wait()
            pltpu.async_copy(rows_b, acc.at[dst_v.at[c1]], s1, add=True)
            sload(c1 + 2, 1, i1)
            pltpu.make_async_copy(rows_a, acc.at[dst_v.at[c0]], s0).wait()
            sload_wait(c0 + 2, 0, i0)
            pltpu.async_copy(g_hbm.at[sidx.at[0]], rows_a, g0)
            pltpu.make_async_copy(rows_b, acc.at[dst_v.at[c1]], s1).wait()
            sload_wait(c1 + 2, 1, i1)
            pltpu.async_copy(g_hbm.at[sidx.at[1]], rows_b, g1)
            return carry

        lax.fori_loop(0, cps // 2 - 1, pair, 0)
        pltpu.make_async_copy(g_hbm.at[sidx.at[0]], rows_a, g0).wait()
        pltpu.sync_copy(rows_a, acc.at[dst_v.at[cps - 2]], add=True)
        pltpu.make_async_copy(g_hbm.at[sidx.at[1]], rows_b, g1).wait()
        pltpu.sync_copy(rows_b, acc.at[dst_v.at[cps - 1]], add=True)
        plsc.subcore_barrier()
        pltpu.sync_copy(acc.at[pl.ds(s * rows_pt, rows_pt), :],
                        out_hbm.at[c, pl.ds(s * rows_pt, rows_pt), :])

    return mp_kernel


def _scale1_body(x_ref, w_ref, degp_ref, out_ref):
    deg = degp_ref[0] + degp_ref[1]
    dinv = lax.rsqrt(jnp.maximum(deg, 1e-12))
    h = jnp.dot(x_ref[...], w_ref[...], preferred_element_type=jnp.float32)
    out_ref[...] = h * dinv


def _layer_mid_body(sp_ref, degp_ref, b_ref, w_ref, out_ref):
    n = degp_ref.shape[1]
    deg = degp_ref[0] + degp_ref[1]
    dinv = lax.rsqrt(jnp.maximum(deg, 1e-12))
    ssum = sp_ref[0, :n, :] + sp_ref[1, :n, :]
    out1 = ssum * dinv + b_ref[...]
    a = jnp.where(out1 > 0, out1, 0.01 * out1)
    h = jnp.dot(a, w_ref[...], preferred_element_type=jnp.float32)
    out_ref[...] = h * dinv


def _final_body(sp_ref, degp_ref, b_ref, out_ref):
    n = out_ref.shape[0]
    deg = degp_ref[0] + degp_ref[1]
    dinv = lax.rsqrt(jnp.maximum(deg, 1e-12))
    ssum = sp_ref[0, :n, :] + sp_ref[1, :n, :]
    out_ref[...] = ssum * dinv + b_ref[...]


def kernel(x, edge_index, W1, b1, W2, b2):
    n, d = x.shape
    e = edge_index.shape[1]
    e_tot = e + n
    cps, e_pad = _edge_plan(e_tot)
    pad = e_pad - e_tot

    loop = jnp.arange(n, dtype=jnp.int32)
    pad_src = jnp.arange(pad, dtype=jnp.int32) % n
    pad_dst = n + (jnp.arange(pad, dtype=jnp.int32) % PAD_ROWS)
    src = jnp.concatenate([edge_index[0], loop, pad_src])
    dst = jnp.concatenate([edge_index[1], loop, pad_dst])
    dst4 = dst.reshape(NS, NC, cps, CH)

    dst_deg = dst.reshape(NS * NC, cps, CH)
    degp = _make_deg_kernel(cps)(dst_deg)
    degp_col = degp[:, :n, None]

    mp = _make_mp_kernel(cps)

    g1 = pl.pallas_call(
        _scale1_body,
        out_shape=jax.ShapeDtypeStruct((n, d), jnp.float32),
    )(x, W1, degp_col)

    s1 = mp(g1, src, dst4)

    g2 = pl.pallas_call(
        _layer_mid_body,
        out_shape=jax.ShapeDtypeStruct((n, d), jnp.float32),
    )(s1, degp_col, b1.reshape(1, d), W2)

    s2 = mp(g2, src, dst4)

    out = pl.pallas_call(
        _final_body,
        out_shape=jax.ShapeDtypeStruct((n, d), jnp.float32),
    )(s2, degp_col, b2.reshape(1, d))
    return out


# speedup vs baseline: 1.2675x; 1.2675x over previous
import functools

import jax
import jax.numpy as jnp
from jax import lax
from jax.experimental import pallas as pl
from jax.experimental.pallas import tpu as pltpu
from jax.experimental.pallas import tpu_sc as plsc

NC = 2
NS = 16
CH = 128
D = 128

N_OUT = 10240
DEG_PAD = N_OUT
PAD_ROWS = 128


def _edge_plan(e_tot):
    cps = -(-e_tot // (NS * NC * CH))
    if cps % 2:
        cps += 1
    e_pad = cps * NS * NC * CH
    return cps, e_pad


def _make_deg_kernel(chalf):
    rows_pt = DEG_PAD // NS
    mesh = plsc.VectorSubcoreMesh(core_axis_name="c", subcore_axis_name="s",
                                  num_cores=NC, num_subcores=NS)

    @functools.partial(
        pl.kernel,
        out_type=jax.ShapeDtypeStruct((NC, DEG_PAD), jnp.float32),
        mesh=mesh,
        scratch_types=[
            pltpu.VMEM((chalf, CH), jnp.int32),
            pltpu.VMEM((CH,), jnp.float32),
            pltpu.VMEM((rows_pt,), jnp.float32),
            pltpu.VMEM_SHARED((DEG_PAD,), jnp.float32),
        ],
    )
    def deg_kernel(dst_hbm, out_hbm, idx_v, ones_v, zb_v, acc):
        c = lax.axis_index("c")
        s = lax.axis_index("s")
        wid = s * NC + c
        one = jnp.ones((16,), jnp.float32)
        zero = jnp.zeros((16,), jnp.float32)
        for i in range(CH // 16):
            ones_v[pl.ds(i * 16, 16)] = one

        def zfill(r, carry):
            zb_v[pl.ds(r * 16, 16)] = zero
            return carry

        lax.fori_loop(0, rows_pt // 16, zfill, 0)
        pltpu.sync_copy(zb_v, acc.at[pl.ds(s * rows_pt, rows_pt)])
        plsc.subcore_barrier()

        pltpu.sync_copy(dst_hbm.at[wid], idx_v)

        def body(j, carry):
            pltpu.sync_copy(ones_v, acc.at[idx_v.at[j]], add=True)
            return carry

        lax.fori_loop(0, chalf, body, 0)
        plsc.subcore_barrier()
        pltpu.sync_copy(acc.at[pl.ds(s * rows_pt, rows_pt)],
                        out_hbm.at[c, pl.ds(s * rows_pt, rows_pt)])

    return deg_kernel


def _make_mp_kernel(cps):
    rows_pt = N_OUT // NS
    epw = cps * CH
    mesh = plsc.VectorSubcoreMesh(core_axis_name="c", subcore_axis_name="s",
                                  num_cores=NC, num_subcores=NS)

    @functools.partial(
        pl.kernel,
        out_type=jax.ShapeDtypeStruct((NC, N_OUT, D), jnp.float32),
        mesh=mesh,
        scratch_types=[
            pltpu.VMEM((cps, CH), jnp.int32),
            pltpu.VMEM((2, CH), jnp.int32),
            pltpu.VMEM((CH, D), jnp.float32),
            pltpu.VMEM((CH, D), jnp.float32),
            pltpu.VMEM_SHARED((N_OUT, D), jnp.float32),
            pltpu.SemaphoreType.DMA,
            pltpu.SemaphoreType.DMA,
            pltpu.SemaphoreType.DMA,
            pltpu.SemaphoreType.DMA,
        ],
    )
    def mp_kernel(g_hbm, src_hbm, dst_hbm, out_hbm,
                  dst_v, sidx, rows_a, rows_b, acc, i0, i1, g0, g1):
        c = lax.axis_index("c")
        s = lax.axis_index("s")
        zero = jnp.zeros((16,), jnp.float32)
        base = (s * NC + c) * epw

        def sload(chunk, slot, sem):
            pltpu.async_copy(src_hbm.at[pl.ds(base + chunk * CH, CH)],
                             sidx.at[slot], sem)

        def sload_wait(chunk, slot, sem):
            pltpu.make_async_copy(src_hbm.at[pl.ds(base + chunk * CH, CH)],
                                  sidx.at[slot], sem).wait()

        sload(0, 0, i0)
        sload(1, 1, i1)
        pltpu.sync_copy(dst_hbm.at[s, c], dst_v)
        sload_wait(1, 1, i1)
        pltpu.async_copy(g_hbm.at[sidx.at[1]], rows_b, g1)

        def zfill(r, carry):
            for i in range(D // 16):
                rows_a[r, pl.ds(i * 16, 16)] = zero
            return carry

        lax.fori_loop(0, CH, zfill, 0)
        for t in range(rows_pt // CH):
            pltpu.sync_copy(rows_a, acc.at[pl.ds(s * rows_pt + t * CH, CH), :])

        sload_wait(0, 0, i0)
        pltpu.async_copy(g_hbm.at[sidx.at[0]], rows_a, g0)
        plsc.subcore_barrier()

        def pair(p, carry):
            c0 = 2 * p
            c1 = c0 + 1
            pltpu.make_async_copy(g_hbm.at[sidx.at[0]], rows_a, g0).wait()
            sload(c0 + 2, 0, i0)
            pltpu.sync_copy(rows_a, acc.at[dst_v.at[c0]], add=True)
            sload_wait(c0 + 2, 0, i0)
            pltpu.async_copy(g_hbm.at[sidx.at[0]], rows_a, g0)
            pltpu.make_async_copy(g_hbm.at[sidx.at[1]], rows_b, g1).wait()
            sload(c1 + 2, 1, i1)
            pltpu.sync_copy(rows_b, acc.at[dst_v.at[c1]], add=True)
            sload_wait(c1 + 2, 1, i1)
            pltpu.async_copy(g_hbm.at[sidx.at[1]], rows_b, g1)
            return carry

        lax.fori_loop(0, cps // 2 - 1, pair, 0)
        pltpu.make_async_copy(g_hbm.at[sidx.at[0]], rows_a, g0).wait()
        pltpu.sync_copy(rows_a, acc.at[dst_v.at[cps - 2]], add=True)
        pltpu.make_async_copy(g_hbm.at[sidx.at[1]], rows_b, g1).wait()
        pltpu.sync_copy(rows_b, acc.at[dst_v.at[cps - 1]], add=True)
        plsc.subcore_barrier()
        pltpu.sync_copy(acc.at[pl.ds(s * rows_pt, rows_pt), :],
                        out_hbm.at[c, pl.ds(s * rows_pt, rows_pt), :])

    return mp_kernel


def _scale1_body(x_ref, w_ref, degp_ref, out_ref):
    deg = degp_ref[0] + degp_ref[1]
    dinv = lax.rsqrt(jnp.maximum(deg, 1e-12))
    h = jnp.dot(x_ref[...], w_ref[...], preferred_element_type=jnp.float32)
    out_ref[...] = h * dinv


def _layer_mid_body(sp_ref, degp_ref, b_ref, w_ref, out_ref):
    n = degp_ref.shape[1]
    deg = degp_ref[0] + degp_ref[1]
    dinv = lax.rsqrt(jnp.maximum(deg, 1e-12))
    ssum = sp_ref[0, :n, :] + sp_ref[1, :n, :]
    out1 = ssum * dinv + b_ref[...]
    a = jnp.where(out1 > 0, out1, 0.01 * out1)
    h = jnp.dot(a, w_ref[...], preferred_element_type=jnp.float32)
    out_ref[...] = h * dinv


def _final_body(sp_ref, degp_ref, b_ref, out_ref):
    n = out_ref.shape[0]
    deg = degp_ref[0] + degp_ref[1]
    dinv = lax.rsqrt(jnp.maximum(deg, 1e-12))
    ssum = sp_ref[0, :n, :] + sp_ref[1, :n, :]
    out_ref[...] = ssum * dinv + b_ref[...]


def kernel(x, edge_index, W1, b1, W2, b2):
    n, d = x.shape
    e = edge_index.shape[1]
    e_tot = e + n
    cps, e_pad = _edge_plan(e_tot)
    pad = e_pad - e_tot

    loop = jnp.arange(n, dtype=jnp.int32)
    pad_src = jnp.arange(pad, dtype=jnp.int32) % n
    pad_dst = n + (jnp.arange(pad, dtype=jnp.int32) % PAD_ROWS)
    src = jnp.concatenate([edge_index[0], loop, pad_src])
    dst = jnp.concatenate([edge_index[1], loop, pad_dst])
    dst4 = dst.reshape(NS, NC, cps, CH)

    dst_deg = dst.reshape(NS * NC, cps, CH)
    degp = _make_deg_kernel(cps)(dst_deg)
    degp_col = degp[:, :n, None]

    mp = _make_mp_kernel(cps)

    g1 = pl.pallas_call(
        _scale1_body,
        out_shape=jax.ShapeDtypeStruct((n, d), jnp.float32),
    )(x, W1, degp_col)

    s1 = mp(g1, src, dst4)

    g2 = pl.pallas_call(
        _layer_mid_body,
        out_shape=jax.ShapeDtypeStruct((n, d), jnp.float32),
    )(s1, degp_col, b1.reshape(1, d), W2)

    s2 = mp(g2, src, dst4)

    out = pl.pallas_call(
        _final_body,
        out_shape=jax.ShapeDtypeStruct((n, d), jnp.float32),
    )(s2, degp_col, b2.reshape(1, d))
    return out
